# Initial kernel scaffold; baseline (speedup 1.0000x reference)
import jax
import jax.numpy as jnp
from jax.experimental import pallas as pl


def kernel(x, edge_index, W):
    raise NotImplementedError("write your pallas kernel here")


# speedup vs baseline: 4.9491x; 4.9491x over previous
import functools

import jax
import jax.numpy as jnp
from jax import lax
from jax.experimental import pallas as pl
from jax.experimental.pallas import tpu as pltpu
from jax.experimental.pallas import tpu_sc as plsc

N = 10000
C = 128
K = 32
NC = 2
NS = 16
NW = NC * NS
NPW = 320
N_PAD = NW * NPW
BATCH = 4
G = NPW // BATCH
RPG = BATCH * K
CB = C // 16

_mesh = plsc.VectorSubcoreMesh(core_axis_name="c", subcore_axis_name="s")


@functools.partial(
    pl.kernel,
    mesh=_mesh,
    out_type=jax.ShapeDtypeStruct((N_PAD, C), jnp.float32),
    scratch_types=[
        pltpu.VMEM((G, RPG), jnp.int32),
        pltpu.VMEM((2, RPG, C), jnp.float32),
        pltpu.VMEM((NPW, C), jnp.float32),
        pltpu.VMEM((K, C), jnp.float32),
        pltpu.SemaphoreType.DMA,
    ],
)
def _cluster_conv(xt_hbm, idx_hbm, wt_hbm, out_hbm,
                  idx_v, rows_v, out_v, wt_v, sem):
    wid = lax.axis_index("s") * NC + lax.axis_index("c")
    pltpu.sync_copy(idx_hbm.at[pl.ds(wid * G, G), :], idx_v)
    pltpu.sync_copy(wt_hbm, wt_v)

    def gbody(g, carry):
        pltpu.async_copy(xt_hbm.at[idx_v.at[g]], rows_v.at[0], sem).wait()

        def kbody(k, accs):
            accs = list(accs)
            for cb in range(CB):
                w = wt_v[k, pl.ds(cb * 16, 16)]
                for n in range(BATCH):
                    r = rows_v[0, n * K + k, pl.ds(cb * 16, 16)]
                    accs[n * CB + cb] = accs[n * CB + cb] + r * w
            return tuple(accs)

        accs = lax.fori_loop(
            0, K, kbody,
            tuple(jnp.zeros((16,), jnp.float32) for _ in range(BATCH * CB)))
        for n in range(BATCH):
            for cb in range(CB):
                out_v[g * BATCH + n, pl.ds(cb * 16, 16)] = accs[n * CB + cb]
        return carry

    lax.fori_loop(0, G, gbody, 0)
    pltpu.sync_copy(out_v, out_hbm.at[pl.ds(wid * NPW, NPW), :])


def kernel(x, edge_index, W):
    xt = x[0].T
    wt = W.T
    idx = edge_index[0].reshape(-1)
    idx = jnp.pad(idx, (0, N_PAD * K - N * K))
    idx2d = idx.reshape(NW * G, RPG)
    out = _cluster_conv(xt, idx2d, wt)
    return out[:N][None]


# speedup vs baseline: 5.3375x; 1.0785x over previous
import functools

import jax
import jax.numpy as jnp
from jax import lax
from jax.experimental import pallas as pl
from jax.experimental.pallas import tpu as pltpu
from jax.experimental.pallas import tpu_sc as plsc

N = 10000
C = 128
K = 32
NC = 2
NS = 16
NW = NC * NS
NPW = 320
N_PAD = NW * NPW
BATCH = 4
G = NPW // BATCH
RPG = BATCH * K
CB = C // 16

_mesh = plsc.VectorSubcoreMesh(core_axis_name="c", subcore_axis_name="s")


IDX_ROWS = NW * G + 8


@functools.partial(
    pl.kernel,
    mesh=_mesh,
    out_type=jax.ShapeDtypeStruct((N_PAD, C), jnp.float32),
    scratch_types=[
        pltpu.VMEM((G + 8, RPG), jnp.int32),
        pltpu.VMEM((2, RPG, C), jnp.float32),
        pltpu.VMEM((NPW, C), jnp.float32),
        pltpu.VMEM((K, C), jnp.float32),
        pltpu.SemaphoreType.DMA,
        pltpu.SemaphoreType.DMA,
    ],
)
def _cluster_conv(xt_hbm, idx_hbm, wt_hbm, out_hbm,
                  idx_v, rows_v, out_v, wt_v, sem0, sem1):
    wid = lax.axis_index("s") * NC + lax.axis_index("c")
    pltpu.sync_copy(idx_hbm.at[pl.ds(wid * G, G + 8), :], idx_v)
    pltpu.sync_copy(wt_hbm, wt_v)
    sems = (sem0, sem1)

    def fire(g, buf):
        pltpu.async_copy(xt_hbm.at[idx_v.at[g]], rows_v.at[buf], sems[buf])

    def wait(g, buf):
        pltpu.make_async_copy(
            xt_hbm.at[idx_v.at[g]], rows_v.at[buf], sems[buf]).wait()

    def compute(g, buf):
        def kbody(k, accs):
            accs = list(accs)
            for cb in range(CB):
                w = wt_v[k, pl.ds(cb * 16, 16)]
                for n in range(BATCH):
                    r = rows_v[buf, n * K + k, pl.ds(cb * 16, 16)]
                    accs[n * CB + cb] = accs[n * CB + cb] + r * w
            return tuple(accs)

        accs = lax.fori_loop(
            0, K, kbody,
            tuple(jnp.zeros((16,), jnp.float32) for _ in range(BATCH * CB)))
        for n in range(BATCH):
            for cb in range(CB):
                out_v[g * BATCH + n, pl.ds(cb * 16, 16)] = accs[n * CB + cb]

    fire(0, 0)

    def pair(p, carry):
        g0 = 2 * p
        wait(g0, 0)
        fire(g0 + 1, 1)
        compute(g0, 0)
        wait(g0 + 1, 1)
        fire(g0 + 2, 0)
        compute(g0 + 1, 1)
        return carry

    lax.fori_loop(0, G // 2, pair, 0)
    wait(G, 0)
    pltpu.sync_copy(out_v, out_hbm.at[pl.ds(wid * NPW, NPW), :])


def kernel(x, edge_index, W):
    xt = x[0].T
    wt = W.T
    idx = edge_index[0].reshape(-1)
    idx = jnp.pad(idx, (0, IDX_ROWS * RPG - N * K))
    idx2d = idx.reshape(IDX_ROWS, RPG)
    out = _cluster_conv(xt, idx2d, wt)
    return out[:N][None]


# speedup vs baseline: 5.8886x; 1.1033x over previous
import functools

import jax
import jax.numpy as jnp
from jax import lax
from jax.experimental import pallas as pl
from jax.experimental.pallas import tpu as pltpu
from jax.experimental.pallas import tpu_sc as plsc

N = 10000
C = 128
K = 32
NC = 2
NS = 16
BATCH = 4
RPG = BATCH * K
CB = C // 16

B0 = 128
B1 = 32
TOT_B = NS * (B0 + B1)
N_PAD = TOT_B * BATCH
IDX_ROWS = TOT_B + 8

_mesh = plsc.VectorSubcoreMesh(core_axis_name="c", subcore_axis_name="s")


@functools.partial(
    pl.kernel,
    mesh=_mesh,
    out_type=jax.ShapeDtypeStruct((N_PAD, C), jnp.float32),
    scratch_types=[
        pltpu.VMEM((B0 + 8, RPG), jnp.int32),
        pltpu.VMEM((2, RPG, C), jnp.float32),
        pltpu.VMEM((B0 * BATCH, C), jnp.float32),
        pltpu.VMEM((K, C), jnp.float32),
        pltpu.SemaphoreType.DMA,
        pltpu.SemaphoreType.DMA,
    ],
)
def _cluster_conv(xt_hbm, idx_hbm, wt_hbm, out_hbm,
                  idx_v, rows_v, out_v, wt_v, sem0, sem1):
    cid = lax.axis_index("c")
    sid = lax.axis_index("s")
    on_c0 = cid == 0
    base_b = jnp.where(on_c0, sid * B0, NS * B0 + sid * B1)
    nb = jnp.where(on_c0, B0, B1)

    @pl.when(on_c0)
    def _():
        pltpu.sync_copy(idx_hbm.at[pl.ds(base_b, B0 + 8), :], idx_v)

    @pl.when(jnp.logical_not(on_c0))
    def _():
        pltpu.sync_copy(idx_hbm.at[pl.ds(base_b, B1 + 8), :],
                        idx_v.at[pl.ds(0, B1 + 8)])

    pltpu.sync_copy(wt_hbm, wt_v)
    sems = (sem0, sem1)

    def fire(g, buf):
        pltpu.async_copy(xt_hbm.at[idx_v.at[g]], rows_v.at[buf], sems[buf])

    def wait(buf):
        pltpu.make_async_copy(
            xt_hbm.at[idx_v.at[0]], rows_v.at[buf], sems[buf]).wait()

    def compute(g, buf):
        def kbody(k, accs):
            accs = list(accs)
            for cb in range(CB):
                w = wt_v[k, pl.ds(cb * 16, 16)]
                for n in range(BATCH):
                    r = rows_v[buf, n * K + k, pl.ds(cb * 16, 16)]
                    accs[n * CB + cb] = accs[n * CB + cb] + r * w
            return tuple(accs)

        accs = lax.fori_loop(
            0, K, kbody,
            tuple(jnp.zeros((16,), jnp.float32) for _ in range(BATCH * CB)))
        for n in range(BATCH):
            for cb in range(CB):
                out_v[g * BATCH + n, pl.ds(cb * 16, 16)] = accs[n * CB + cb]

    fire(0, 0)

    def pair(p, carry):
        g0 = 2 * p
        wait(0)
        fire(g0 + 1, 1)
        compute(g0, 0)
        wait(1)
        fire(g0 + 2, 0)
        compute(g0 + 1, 1)
        return carry

    lax.fori_loop(0, nb // 2, pair, 0, unroll=False)
    wait(0)

    @pl.when(on_c0)
    def _():
        pltpu.sync_copy(out_v,
                        out_hbm.at[pl.ds(base_b * BATCH, B0 * BATCH), :])

    @pl.when(jnp.logical_not(on_c0))
    def _():
        pltpu.sync_copy(out_v.at[pl.ds(0, B1 * BATCH)],
                        out_hbm.at[pl.ds(base_b * BATCH, B1 * BATCH), :])


def kernel(x, edge_index, W):
    xt = x[0].T
    wt = W.T
    idx = edge_index[0].reshape(-1)
    idx = jnp.pad(idx, (0, IDX_ROWS * RPG - N * K))
    idx2d = idx.reshape(IDX_ROWS, RPG)
    out = _cluster_conv(xt, idx2d, wt)
    return out[:N][None]
